# baseline (speedup 1.0000x reference)
import jax
import jax.numpy as jnp
import numpy as np
from jax.experimental import pallas as pl
from jax.experimental.pallas import tpu as pltpu

D = 768
S = 1024
H = 12
HD = 64
KW = 31
PAD = 15
FF = 3072
NG = 2
NE = 2
JB = FF // D
LN_EPS = 1e-6


def _ln(x, scale, bias):
    m = jnp.mean(x, axis=-1, keepdims=True)
    v = jnp.mean((x - m) ** 2, axis=-1, keepdims=True)
    return (x - m) * jax.lax.rsqrt(v + LN_EPS) * scale + bias


def _mm(a, b):
    return jnp.dot(a.astype(jnp.bfloat16), b.astype(jnp.bfloat16),
                   preferred_element_type=jnp.float32)


SPAD = S + 32


def _conv_body(x_ref, w_ref, b_ref, sc_ref, bi_ref, o_ref, xpad8_ref):
    k = pl.program_id(0)

    @pl.when(k == 0)
    def _init():
        xn = _ln(x_ref[...], sc_ref[...], bi_ref[...]).astype(jnp.bfloat16)
        ext = jnp.concatenate([xn, jnp.zeros((SPAD - S, D), jnp.bfloat16)],
                              axis=0)
        for r in range(8):
            xpad8_ref[r] = jnp.roll(ext, PAD - r, axis=0)
        o_ref[...] = jnp.zeros_like(o_ref)

    q = pl.multiple_of(8 * (k // 8), 8)
    o_ref[...] += _mm(xpad8_ref[k % 8, pl.ds(q, S), :], w_ref[0])

    @pl.when(k == KW - 1)
    def _fin():
        o_ref[...] = jax.nn.gelu(o_ref[...] + b_ref[...]) + x_ref[...]


def _conv_block(x, p, lnp):
    return pl.pallas_call(
        _conv_body,
        grid=(KW,),
        in_specs=[
            pl.BlockSpec((S, D), lambda k: (0, 0)),
            pl.BlockSpec((1, D, D), lambda k: (k, 0, 0)),
            pl.BlockSpec((1, D), lambda k: (0, 0)),
            pl.BlockSpec((1, D), lambda k: (0, 0)),
            pl.BlockSpec((1, D), lambda k: (0, 0)),
        ],
        out_specs=pl.BlockSpec((S, D), lambda k: (0, 0)),
        out_shape=jax.ShapeDtypeStruct((S, D), jnp.float32),
        scratch_shapes=[pltpu.VMEM((8, SPAD, D), jnp.bfloat16)],
        compiler_params=pltpu.CompilerParams(
            dimension_semantics=("arbitrary",)),
    )(x, p["w"], p["b"].reshape(1, D), lnp["scale"].reshape(1, D),
      lnp["bias"].reshape(1, D))


HB = 128
HPB = HB // HD


def _attn_body(x_ref, wq_ref, bq_ref, wk_ref, bk_ref, wv_ref, bv_ref,
               wo_ref, bo_ref, sc_ref, bi_ref, o_ref, q_ref, k_ref, v_ref):
    step = pl.program_id(0)

    @pl.when(step == 0)
    def _init():
        xn = _ln(x_ref[...], sc_ref[...], bi_ref[...])
        qscale = 1.0 / np.sqrt(HD).astype(np.float32)
        q_ref[...] = ((_mm(xn, wq_ref[...]) + bq_ref[...]) *
                      qscale).astype(jnp.bfloat16)
        k_ref[...] = (_mm(xn, wk_ref[...]) + bk_ref[...]).astype(jnp.bfloat16)
        v_ref[...] = (_mm(xn, wv_ref[...]) + bv_ref[...]).astype(jnp.bfloat16)
        o_ref[...] = x_ref[...] + bo_ref[...]

    @pl.when(step > 0)
    def _heads():
        hb = step - 1
        lo = pl.multiple_of(hb * HB, HB)
        qb = q_ref[:, pl.ds(lo, HB)]
        kb = k_ref[:, pl.ds(lo, HB)]
        vb = v_ref[:, pl.ds(lo, HB)]
        outs = []
        for i in range(HPB):
            qi = qb[:, i * HD:(i + 1) * HD]
            ki = kb[:, i * HD:(i + 1) * HD]
            vi = vb[:, i * HD:(i + 1) * HD]
            logits = _mm(qi, ki.T)
            mx = jnp.max(logits, axis=-1, keepdims=True)
            e = jnp.exp(logits - mx)
            z = jnp.sum(e, axis=-1, keepdims=True)
            outs.append(_mm(e, vi) * (1.0 / z))
        o_ref[...] += _mm(jnp.concatenate(outs, axis=-1), wo_ref[...])


def _attn_block(x, p, lnp):
    nhb = H // HPB
    return pl.pallas_call(
        _attn_body,
        grid=(nhb + 1,),
        in_specs=[
            pl.BlockSpec((S, D), lambda s: (0, 0)),
            pl.BlockSpec((D, D), lambda s: (0, 0)),
            pl.BlockSpec((1, D), lambda s: (0, 0)),
            pl.BlockSpec((D, D), lambda s: (0, 0)),
            pl.BlockSpec((1, D), lambda s: (0, 0)),
            pl.BlockSpec((D, D), lambda s: (0, 0)),
            pl.BlockSpec((1, D), lambda s: (0, 0)),
            pl.BlockSpec((HB, D), lambda s: (jnp.maximum(s - 1, 0), 0)),
            pl.BlockSpec((1, D), lambda s: (0, 0)),
            pl.BlockSpec((1, D), lambda s: (0, 0)),
            pl.BlockSpec((1, D), lambda s: (0, 0)),
        ],
        out_specs=pl.BlockSpec((S, D), lambda s: (0, 0)),
        out_shape=jax.ShapeDtypeStruct((S, D), jnp.float32),
        scratch_shapes=[pltpu.VMEM((S, D), jnp.bfloat16),
                        pltpu.VMEM((S, D), jnp.bfloat16),
                        pltpu.VMEM((S, D), jnp.bfloat16)],
        compiler_params=pltpu.CompilerParams(
            dimension_semantics=("arbitrary",)),
    )(x, p["q"]["w"], p["q"]["b"].reshape(1, D),
      p["k"]["w"], p["k"]["b"].reshape(1, D),
      p["v"]["w"], p["v"]["b"].reshape(1, D),
      p["o"]["w"], p["o"]["b"].reshape(1, D),
      lnp["scale"].reshape(1, D), lnp["bias"].reshape(1, D))


def _ff_body(x_ref, w1_ref, b1_ref, w2_ref, b2_ref, sc_ref, bi_ref,
             o_ref, xn_ref):
    j = pl.program_id(0)

    @pl.when(j == 0)
    def _init():
        xn_ref[...] = _ln(x_ref[...], sc_ref[...], bi_ref[...])
        o_ref[...] = x_ref[...] + b2_ref[...]

    hidden = jax.nn.gelu(_mm(xn_ref[...], w1_ref[...]) + b1_ref[...])
    o_ref[...] += _mm(hidden, w2_ref[...])


def _ff_block(x, p, lnp):
    return pl.pallas_call(
        _ff_body,
        grid=(JB,),
        in_specs=[
            pl.BlockSpec((S, D), lambda j: (0, 0)),
            pl.BlockSpec((D, D), lambda j: (0, j)),
            pl.BlockSpec((1, D), lambda j: (0, j)),
            pl.BlockSpec((D, D), lambda j: (j, 0)),
            pl.BlockSpec((1, D), lambda j: (0, 0)),
            pl.BlockSpec((1, D), lambda j: (0, 0)),
            pl.BlockSpec((1, D), lambda j: (0, 0)),
        ],
        out_specs=pl.BlockSpec((S, D), lambda j: (0, 0)),
        out_shape=jax.ShapeDtypeStruct((S, D), jnp.float32),
        scratch_shapes=[pltpu.VMEM((S, D), jnp.float32)],
        compiler_params=pltpu.CompilerParams(
            dimension_semantics=("arbitrary",)),
    )(x, p["ff1"]["w"], p["ff1"]["b"].reshape(1, FF),
      p["ff2"]["w"], p["ff2"]["b"].reshape(1, D),
      lnp["scale"].reshape(1, D), lnp["bias"].reshape(1, D))


def _moe_body(x_ref, gid_ref, w1_ref, b1_ref, w2_ref, b2_ref, o_ref,
              gacc_ref):
    g = pl.program_id(0)
    e = pl.program_id(1)
    j = pl.program_id(2)

    @pl.when((g == 0) & (e == 0) & (j == 0))
    def _init_out():
        o_ref[...] = x_ref[...]

    @pl.when((e == 0) & (j == 0))
    def _init_group():
        gacc_ref[...] = jnp.zeros_like(gacc_ref)

    @pl.when(j == 0)
    def _bias2():
        gacc_ref[...] += b2_ref[0]

    hidden = jax.nn.gelu(_mm(x_ref[...], w1_ref[0]) + b1_ref[0])
    gacc_ref[...] += _mm(hidden, w2_ref[0])

    @pl.when((e == NE - 1) & (j == JB - 1))
    def _write():
        mask = gid_ref[...] == g
        o_ref[...] = jnp.where(
            mask, x_ref[...] + (1.0 / NE) * gacc_ref[...], o_ref[...])


def _moe_block(x, gids, expert_groups):
    w1 = jnp.stack([ep["fc1"]["w"] for grp in expert_groups for ep in grp])
    b1 = jnp.stack([ep["fc1"]["b"].reshape(1, FF)
                    for grp in expert_groups for ep in grp])
    w2 = jnp.stack([ep["fc2"]["w"] for grp in expert_groups for ep in grp])
    b2 = jnp.stack([ep["fc2"]["b"].reshape(1, D)
                    for grp in expert_groups for ep in grp])
    return pl.pallas_call(
        _moe_body,
        grid=(NG, NE, JB),
        in_specs=[
            pl.BlockSpec((S, D), lambda g, e, j: (0, 0)),
            pl.BlockSpec((S, 1), lambda g, e, j: (0, 0)),
            pl.BlockSpec((1, D, D), lambda g, e, j: (g * NE + e, 0, j)),
            pl.BlockSpec((1, 1, D), lambda g, e, j: (g * NE + e, 0, j)),
            pl.BlockSpec((1, D, D), lambda g, e, j: (g * NE + e, j, 0)),
            pl.BlockSpec((1, 1, D), lambda g, e, j: (g * NE + e, 0, 0)),
        ],
        out_specs=pl.BlockSpec((S, D), lambda g, e, j: (0, 0)),
        out_shape=jax.ShapeDtypeStruct((S, D), jnp.float32),
        scratch_shapes=[pltpu.VMEM((S, D), jnp.float32)],
        compiler_params=pltpu.CompilerParams(
            dimension_semantics=("arbitrary", "arbitrary", "arbitrary")),
    )(x, gids.reshape(S, 1), w1, b1, w2, b2)


def kernel(x, group_ids, params):
    b, s, d = x.shape
    xs = x.reshape(S, D)
    gids = group_ids.reshape(S)
    for i, p in enumerate(params["layers"]):
        is_moe = 1 <= i < 2
        xs = _conv_block(xs, p["conv"], p["ln1"])
        xs = _attn_block(xs, p["attn"], p["ln2"])
        if is_moe:
            xs = _moe_block(xs, gids, params["expert_groups"])
        else:
            xs = _ff_block(xs, p, p["ln3"])
    return xs.reshape(b, s, d)


# speedup vs baseline: 1.0034x; 1.0034x over previous
import jax
import jax.numpy as jnp
import numpy as np
from jax.experimental import pallas as pl
from jax.experimental.pallas import tpu as pltpu

D = 768
S = 1024
H = 12
HD = 64
KW = 31
PAD = 15
FF = 3072
NG = 2
NE = 2
JB = FF // D
LN_EPS = 1e-6


def _ln(x, scale, bias):
    m = jnp.mean(x, axis=-1, keepdims=True)
    v = jnp.mean((x - m) ** 2, axis=-1, keepdims=True)
    return (x - m) * jax.lax.rsqrt(v + LN_EPS) * scale + bias


def _mm(a, b):
    return jnp.dot(a.astype(jnp.bfloat16), b.astype(jnp.bfloat16),
                   preferred_element_type=jnp.float32)


SPAD = S + 32


def _conv_body(x_ref, w_ref, b_ref, sc_ref, bi_ref, o_ref, xpad8_ref):
    k = pl.program_id(0)

    @pl.when(k == 0)
    def _init():
        xn = _ln(x_ref[...], sc_ref[...], bi_ref[...])
        ext = jnp.concatenate([xn, jnp.zeros((SPAD - S, D), jnp.float32)],
                              axis=0)
        for r in range(8):
            xpad8_ref[r] = jnp.roll(ext, PAD - r, axis=0)
        o_ref[...] = jnp.zeros_like(o_ref)

    q = pl.multiple_of(8 * (k // 8), 8)
    o_ref[...] += _mm(xpad8_ref[k % 8, pl.ds(q, S), :], w_ref[0])

    @pl.when(k == KW - 1)
    def _fin():
        o_ref[...] = jax.nn.gelu(o_ref[...] + b_ref[...]) + x_ref[...]


def _conv_block(x, p, lnp):
    return pl.pallas_call(
        _conv_body,
        grid=(KW,),
        in_specs=[
            pl.BlockSpec((S, D), lambda k: (0, 0)),
            pl.BlockSpec((1, D, D), lambda k: (k, 0, 0)),
            pl.BlockSpec((1, D), lambda k: (0, 0)),
            pl.BlockSpec((1, D), lambda k: (0, 0)),
            pl.BlockSpec((1, D), lambda k: (0, 0)),
        ],
        out_specs=pl.BlockSpec((S, D), lambda k: (0, 0)),
        out_shape=jax.ShapeDtypeStruct((S, D), jnp.float32),
        scratch_shapes=[pltpu.VMEM((8, SPAD, D), jnp.float32)],
        compiler_params=pltpu.CompilerParams(
            dimension_semantics=("arbitrary",)),
    )(x, p["w"], p["b"].reshape(1, D), lnp["scale"].reshape(1, D),
      lnp["bias"].reshape(1, D))


HB = 128
HPB = HB // HD


def _attn_body(x_ref, wq_ref, bq_ref, wk_ref, bk_ref, wv_ref, bv_ref,
               wo_ref, bo_ref, sc_ref, bi_ref, o_ref, q_ref, k_ref, v_ref):
    step = pl.program_id(0)

    @pl.when(step == 0)
    def _init():
        xn = _ln(x_ref[...], sc_ref[...], bi_ref[...])
        qscale = 1.0 / np.sqrt(HD).astype(np.float32)
        q_ref[...] = ((_mm(xn, wq_ref[...]) + bq_ref[...]) *
                      qscale).astype(jnp.bfloat16)
        k_ref[...] = (_mm(xn, wk_ref[...]) + bk_ref[...]).astype(jnp.bfloat16)
        v_ref[...] = (_mm(xn, wv_ref[...]) + bv_ref[...]).astype(jnp.bfloat16)
        o_ref[...] = x_ref[...] + bo_ref[...]

    @pl.when(step > 0)
    def _heads():
        hb = step - 1
        lo = pl.multiple_of(hb * HB, HB)
        qb = q_ref[:, pl.ds(lo, HB)]
        kb = k_ref[:, pl.ds(lo, HB)]
        vb = v_ref[:, pl.ds(lo, HB)]
        outs = []
        for i in range(HPB):
            qi = qb[:, i * HD:(i + 1) * HD]
            ki = kb[:, i * HD:(i + 1) * HD]
            vi = vb[:, i * HD:(i + 1) * HD]
            logits = _mm(qi, ki.T)
            mx = jnp.max(logits, axis=-1, keepdims=True)
            e = jnp.exp(logits - mx)
            z = jnp.sum(e, axis=-1, keepdims=True)
            outs.append(_mm(e, vi) * (1.0 / z))
        o_ref[...] += _mm(jnp.concatenate(outs, axis=-1), wo_ref[...])


def _attn_block(x, p, lnp):
    nhb = H // HPB
    return pl.pallas_call(
        _attn_body,
        grid=(nhb + 1,),
        in_specs=[
            pl.BlockSpec((S, D), lambda s: (0, 0)),
            pl.BlockSpec((D, D), lambda s: (0, 0)),
            pl.BlockSpec((1, D), lambda s: (0, 0)),
            pl.BlockSpec((D, D), lambda s: (0, 0)),
            pl.BlockSpec((1, D), lambda s: (0, 0)),
            pl.BlockSpec((D, D), lambda s: (0, 0)),
            pl.BlockSpec((1, D), lambda s: (0, 0)),
            pl.BlockSpec((HB, D), lambda s: (jnp.maximum(s - 1, 0), 0)),
            pl.BlockSpec((1, D), lambda s: (0, 0)),
            pl.BlockSpec((1, D), lambda s: (0, 0)),
            pl.BlockSpec((1, D), lambda s: (0, 0)),
        ],
        out_specs=pl.BlockSpec((S, D), lambda s: (0, 0)),
        out_shape=jax.ShapeDtypeStruct((S, D), jnp.float32),
        scratch_shapes=[pltpu.VMEM((S, D), jnp.bfloat16),
                        pltpu.VMEM((S, D), jnp.bfloat16),
                        pltpu.VMEM((S, D), jnp.bfloat16)],
        compiler_params=pltpu.CompilerParams(
            dimension_semantics=("arbitrary",)),
    )(x, p["q"]["w"], p["q"]["b"].reshape(1, D),
      p["k"]["w"], p["k"]["b"].reshape(1, D),
      p["v"]["w"], p["v"]["b"].reshape(1, D),
      p["o"]["w"], p["o"]["b"].reshape(1, D),
      lnp["scale"].reshape(1, D), lnp["bias"].reshape(1, D))


def _ff_body(x_ref, w1_ref, b1_ref, w2_ref, b2_ref, sc_ref, bi_ref,
             o_ref, xn_ref):
    j = pl.program_id(0)

    @pl.when(j == 0)
    def _init():
        xn_ref[...] = _ln(x_ref[...], sc_ref[...], bi_ref[...])
        o_ref[...] = x_ref[...] + b2_ref[...]

    hidden = jax.nn.gelu(_mm(xn_ref[...], w1_ref[...]) + b1_ref[...])
    o_ref[...] += _mm(hidden, w2_ref[...])


def _ff_block(x, p, lnp):
    return pl.pallas_call(
        _ff_body,
        grid=(JB,),
        in_specs=[
            pl.BlockSpec((S, D), lambda j: (0, 0)),
            pl.BlockSpec((D, D), lambda j: (0, j)),
            pl.BlockSpec((1, D), lambda j: (0, j)),
            pl.BlockSpec((D, D), lambda j: (j, 0)),
            pl.BlockSpec((1, D), lambda j: (0, 0)),
            pl.BlockSpec((1, D), lambda j: (0, 0)),
            pl.BlockSpec((1, D), lambda j: (0, 0)),
        ],
        out_specs=pl.BlockSpec((S, D), lambda j: (0, 0)),
        out_shape=jax.ShapeDtypeStruct((S, D), jnp.float32),
        scratch_shapes=[pltpu.VMEM((S, D), jnp.float32)],
        compiler_params=pltpu.CompilerParams(
            dimension_semantics=("arbitrary",)),
    )(x, p["ff1"]["w"], p["ff1"]["b"].reshape(1, FF),
      p["ff2"]["w"], p["ff2"]["b"].reshape(1, D),
      lnp["scale"].reshape(1, D), lnp["bias"].reshape(1, D))


def _moe_body(x_ref, gid_ref, w1_ref, b1_ref, w2_ref, b2_ref, o_ref,
              gacc_ref):
    g = pl.program_id(0)
    e = pl.program_id(1)
    j = pl.program_id(2)

    @pl.when((g == 0) & (e == 0) & (j == 0))
    def _init_out():
        o_ref[...] = x_ref[...]

    @pl.when((e == 0) & (j == 0))
    def _init_group():
        gacc_ref[...] = jnp.zeros_like(gacc_ref)

    @pl.when(j == 0)
    def _bias2():
        gacc_ref[...] += b2_ref[0]

    hidden = jax.nn.gelu(_mm(x_ref[...], w1_ref[0]) + b1_ref[0])
    gacc_ref[...] += _mm(hidden, w2_ref[0])

    @pl.when((e == NE - 1) & (j == JB - 1))
    def _write():
        mask = gid_ref[...] == g
        o_ref[...] = jnp.where(
            mask, x_ref[...] + (1.0 / NE) * gacc_ref[...], o_ref[...])


def _moe_block(x, gids, expert_groups):
    w1 = jnp.stack([ep["fc1"]["w"] for grp in expert_groups for ep in grp])
    b1 = jnp.stack([ep["fc1"]["b"].reshape(1, FF)
                    for grp in expert_groups for ep in grp])
    w2 = jnp.stack([ep["fc2"]["w"] for grp in expert_groups for ep in grp])
    b2 = jnp.stack([ep["fc2"]["b"].reshape(1, D)
                    for grp in expert_groups for ep in grp])
    return pl.pallas_call(
        _moe_body,
        grid=(NG, NE, JB),
        in_specs=[
            pl.BlockSpec((S, D), lambda g, e, j: (0, 0)),
            pl.BlockSpec((S, 1), lambda g, e, j: (0, 0)),
            pl.BlockSpec((1, D, D), lambda g, e, j: (g * NE + e, 0, j)),
            pl.BlockSpec((1, 1, D), lambda g, e, j: (g * NE + e, 0, j)),
            pl.BlockSpec((1, D, D), lambda g, e, j: (g * NE + e, j, 0)),
            pl.BlockSpec((1, 1, D), lambda g, e, j: (g * NE + e, 0, 0)),
        ],
        out_specs=pl.BlockSpec((S, D), lambda g, e, j: (0, 0)),
        out_shape=jax.ShapeDtypeStruct((S, D), jnp.float32),
        scratch_shapes=[pltpu.VMEM((S, D), jnp.float32)],
        compiler_params=pltpu.CompilerParams(
            dimension_semantics=("arbitrary", "arbitrary", "arbitrary")),
    )(x, gids.reshape(S, 1), w1, b1, w2, b2)


def kernel(x, group_ids, params):
    b, s, d = x.shape
    xs = x.reshape(S, D)
    gids = group_ids.reshape(S)
    for i, p in enumerate(params["layers"]):
        is_moe = 1 <= i < 2
        xs = _conv_block(xs, p["conv"], p["ln1"])
        xs = _attn_block(xs, p["attn"], p["ln2"])
        if is_moe:
            xs = _moe_block(xs, gids, params["expert_groups"])
        else:
            xs = _ff_block(xs, p, p["ln3"])
    return xs.reshape(b, s, d)
